# baseline (speedup 1.0000x reference)
import functools

import jax
import jax.numpy as jnp
from jax import lax
from jax.experimental import pallas as pl
from jax.experimental.pallas import tpu as pltpu
from jax.experimental.pallas import tpu_sc as plsc

D = 64
INNER = 16
N_TRAIN = 64
ADJ_ROWS = 72

NC, NS, LANES = 2, 16, 16
NW = NC * NS
SUB = 64
KT = 32


def _adj_table_kernel(t_ref, aw_ref, ab_ref, bw_ref, bb_ref, out_ref):
    t = t_ref[...]
    a = lax.dot_general(t, aw_ref[...], (((1,), (1,)), ((), ())),
                        preferred_element_type=jnp.float32)
    a = a + ab_ref[...]
    adj = lax.dot_general(a, bw_ref[...], (((1,), (1,)), ((), ())),
                          preferred_element_type=jnp.float32)
    adj = adj + bb_ref[...]
    out_ref[...] = jnp.zeros_like(out_ref)
    out_ref[0:N_TRAIN, :] = adj


def _compute_adj_table(trainable_table, A_w, A_b, B_w, B_b):
    return pl.pallas_call(
        _adj_table_kernel,
        out_shape=jax.ShapeDtypeStruct((ADJ_ROWS, D), jnp.float32),
    )(trainable_table, A_w, A_b.reshape(1, INNER), B_w, B_b.reshape(1, D))


def _make_depad(vocab):
    n_tiles = vocab // 8
    per_w = (n_tiles // (NW * 2 * KT)) * (2 * KT)
    tail = n_tiles - per_w * NW
    n_blk = per_w // KT
    assert n_blk * KT == per_w and n_blk % 2 == 0
    assert tail % 8 == 0

    mesh = plsc.VectorSubcoreMesh(core_axis_name="c", subcore_axis_name="s")

    @functools.partial(
        pl.kernel,
        out_type=jax.ShapeDtypeStruct((vocab // 2, 128), jnp.float32),
        mesh=mesh,
        scratch_types=[
            pltpu.VMEM((KT, 8, D), jnp.float32),
            pltpu.VMEM((KT, 8, D), jnp.float32),
            pltpu.VMEM((4 * KT, 128), jnp.float32),
            pltpu.VMEM((4 * KT, 128), jnp.float32),
            pltpu.VMEM((8, 8, D), jnp.float32),
            pltpu.SemaphoreType.DMA,
            pltpu.SemaphoreType.DMA,
            pltpu.SemaphoreType.DMA,
            pltpu.SemaphoreType.DMA,
        ],
    )
    def depad(table2d_hbm, out_hbm, va, vb, vpa, vpb, vt,
              sem_a, sem_b, sem_wa, sem_wb):
        wid = lax.axis_index("s") * NC + lax.axis_index("c")
        table_hbm = table2d_hbm.reshape(n_tiles, 8, D)
        t0 = wid * per_w

        def read(blk, dst, sem):
            pltpu.async_copy(table_hbm.at[pl.ds(t0 + blk * KT, KT)], dst, sem)

        def drain_r(dst, sem):
            pltpu.make_async_copy(table_hbm.at[pl.ds(0, KT)], dst, sem).wait()

        def drain_w(vp, sem):
            pltpu.make_async_copy(out_hbm.at[pl.ds(0, 4 * KT)], vp,
                                  sem).wait()

        def repack(src, vp):
            for t in range(KT):
                for s in range(8):
                    for c in range(0, D, LANES):
                        v = src[t, s, pl.ds(c, LANES)]
                        flat = (t * 8 + s) * D + c
                        vp[flat // 128, pl.ds(flat % 128, LANES)] = v

        def write(blk, vp, sem):
            pltpu.async_copy(
                vp, out_hbm.at[pl.ds((t0 + blk * KT) * 4, 4 * KT)], sem)

        read(0, va, sem_a)

        def pair_body(i, carry):
            blk = i * 2
            read(blk + 1, vb, sem_b)
            drain_r(va, sem_a)

            @pl.when(i > 0)
            def _():
                drain_w(vpa, sem_wa)
            repack(va, vpa)
            write(blk, vpa, sem_wa)

            @pl.when(blk + 2 < n_blk)
            def _():
                read(blk + 2, va, sem_a)
            drain_r(vb, sem_b)

            @pl.when(i > 0)
            def _():
                drain_w(vpb, sem_wb)
            repack(vb, vpb)
            write(blk + 1, vpb, sem_wb)
            return carry

        lax.fori_loop(0, n_blk // 2, pair_body, 0)
        drain_w(vpa, sem_wa)
        drain_w(vpb, sem_wb)

        if tail:
            @pl.when(wid == 0)
            def _tail():
                base = n_tiles - tail

                def tail_body(tb, carry):
                    pltpu.sync_copy(table_hbm.at[pl.ds(base + tb * 8, 8)], vt)
                    for t in range(8):
                        for s in range(8):
                            for c in range(0, D, LANES):
                                v = vt[t, s, pl.ds(c, LANES)]
                                flat = (t * 8 + s) * D + c
                                vpa[flat // 128, pl.ds(flat % 128, LANES)] = v
                    pltpu.sync_copy(vpa.at[pl.ds(0, 32)],
                                    out_hbm.at[pl.ds((base + tb * 8) * 4, 32)])
                    return carry

                lax.fori_loop(0, tail // 8, tail_body, 0)

    return depad


def _make_sc_lookup(n_total, vocab, out_shape3):
    assert n_total % (NW * SUB) == 0
    per_w = n_total // NW
    chunk = 320
    assert per_w % chunk == 0 and chunk % SUB == 0
    n_chunks = per_w // chunk
    nsub = chunk // SUB

    mesh = plsc.VectorSubcoreMesh(core_axis_name="c", subcore_axis_name="s")

    @functools.partial(
        pl.kernel,
        out_type=jax.ShapeDtypeStruct(out_shape3, jnp.float32),
        mesh=mesh,
        scratch_types=[
            pltpu.VMEM((chunk,), jnp.int32),
            pltpu.VMEM((chunk,), jnp.int32),
            pltpu.VMEM((SUB, 128), jnp.float32),
            pltpu.VMEM((SUB, 128), jnp.float32),
            pltpu.VMEM((SUB, 128), jnp.float32),
            pltpu.VMEM((SUB, 128), jnp.float32),
            pltpu.VMEM((chunk // 8, 8, D), jnp.float32),
            pltpu.VMEM((ADJ_ROWS, D), jnp.float32),
            pltpu.SemaphoreType.DMA,
            pltpu.SemaphoreType.DMA,
            pltpu.SemaphoreType.DMA,
            pltpu.SemaphoreType.DMA,
        ],
        compiler_params=pltpu.CompilerParams(needs_layout_passes=False),
    )
    def sc_lookup(ids_hbm, pairs_hbm, adj_hbm, out3_hbm,
                  idx_v, pidx_v, pva, pvb, pvc, pvd, rows_v, adj_v,
                  sem_a, sem_b, sem_c, sem_d):
        wid = lax.axis_index("s") * NC + lax.axis_index("c")
        out_hbm = out3_hbm.reshape(n_total // 8, 8, D)
        pltpu.sync_copy(adj_hbm, adj_v)
        base_w = wid * per_w
        lane = jnp.arange(LANES, dtype=jnp.int32)

        def chunk_body(ch, carry):
            base = base_w + ch * chunk
            pltpu.sync_copy(ids_hbm.at[pl.ds(base, chunk)], idx_v)
            for q in range(chunk // LANES):
                iv = idx_v[pl.ds(q * LANES, LANES)]
                pidx_v[pl.ds(q * LANES, LANES)] = jnp.right_shift(
                    iv, jnp.int32(1))
            bufs = [(pva, sem_a), (pvb, sem_b), (pvc, sem_c), (pvd, sem_d)]

            def fire(j):
                buf, sem = bufs[j % 4]
                return pltpu.async_copy(
                    pairs_hbm.at[pidx_v.at[pl.ds(j * SUB, SUB)]], buf, sem)

            cps = [fire(0), fire(1), fire(2), fire(3)]
            sub_cnt = []
            for j in range(nsub):
                mn = idx_v[pl.ds(j * SUB, LANES)]
                for k in range(1, SUB // LANES):
                    mn = jnp.minimum(mn, idx_v[pl.ds(j * SUB + k * LANES,
                                                     LANES)])
                sub_cnt.append(jnp.sum(jnp.minimum(mn, jnp.int32(N_TRAIN))))

            for j in range(nsub):
                cps[j].wait()
                buf = bufs[j % 4][0]

                def egroup(k, _, j=j, buf=buf):
                    p0 = j * SUB + k * LANES
                    iv = idx_v[pl.ds(p0, LANES)]
                    half = jnp.bitwise_and(iv, jnp.int32(1)) * D
                    slot = k * LANES + lane
                    pv = p0 + lane
                    prow = jnp.right_shift(pv, jnp.int32(3))
                    psub = jnp.bitwise_and(pv, jnp.int32(7))
                    for c in range(D):
                        cvec = jnp.full((LANES,), c, jnp.int32)
                        vals = plsc.load_gather(buf, [slot, half + cvec])
                        plsc.store_scatter(rows_v, [prow, psub, cvec], vals)
                    return 0
                lax.fori_loop(0, SUB // LANES, egroup, 0)
                if j + 4 < nsub:
                    cps.append(fire(j + 4))

            for j in range(nsub):
                @pl.when(sub_cnt[j] < LANES * N_TRAIN)
                def _fixup(j=j):
                    def body(k, _):
                        r0 = j * SUB + k * LANES
                        iv = idx_v[pl.ds(r0, LANES)]
                        clamped = jnp.minimum(iv, jnp.int32(N_TRAIN))
                        rv = r0 + lane
                        rrow = jnp.right_shift(rv, jnp.int32(3))
                        rsub = jnp.bitwise_and(rv, jnp.int32(7))
                        for c in range(D):
                            cvec = jnp.full((LANES,), c, jnp.int32)
                            a = plsc.load_gather(adj_v, [clamped, cvec])
                            r = plsc.load_gather(rows_v, [rrow, rsub, cvec])
                            plsc.store_scatter(rows_v, [rrow, rsub, cvec],
                                               r + a)
                        return 0
                    lax.fori_loop(0, SUB // LANES, body, 0)

            pltpu.sync_copy(rows_v,
                            out_hbm.at[pl.ds(base // 8, chunk // 8)])
            return 0

        lax.fori_loop(0, n_chunks, chunk_body, 0)

    return sc_lookup


def kernel(input_ids, ids, orig_table, trainable_table, A_w, A_b, B_w, B_b):
    b, l = input_ids.shape
    n_total = b * l
    vocab = orig_table.shape[0]
    adj = _compute_adj_table(trainable_table, A_w, A_b, B_w, B_b)
    ids_flat = input_ids.astype(jnp.int32).reshape(n_total)
    pairs = _make_depad(vocab)(orig_table)
    out = _make_sc_lookup(n_total, vocab, (b, l, D))(ids_flat, pairs, adj)
    return out


# speedup vs baseline: 1.4392x; 1.4392x over previous
import functools

import jax
import jax.numpy as jnp
from jax import lax
from jax.experimental import pallas as pl
from jax.experimental.pallas import tpu as pltpu
from jax.experimental.pallas import tpu_sc as plsc

D = 64
INNER = 16
N_TRAIN = 64
ADJ_ROWS = 72

NC, NS, LANES = 2, 16, 16
NW = NC * NS
SUB = 64
KT = 32


def _adj_table_kernel(t_ref, aw_ref, ab_ref, bw_ref, bb_ref, out_ref):
    t = t_ref[...]
    a = lax.dot_general(t, aw_ref[...], (((1,), (1,)), ((), ())),
                        preferred_element_type=jnp.float32)
    a = a + ab_ref[...]
    adj = lax.dot_general(a, bw_ref[...], (((1,), (1,)), ((), ())),
                          preferred_element_type=jnp.float32)
    adj = adj + bb_ref[...]
    out_ref[...] = jnp.zeros_like(out_ref)
    out_ref[0:N_TRAIN, :] = adj


def _compute_adj_table(trainable_table, A_w, A_b, B_w, B_b):
    return pl.pallas_call(
        _adj_table_kernel,
        out_shape=jax.ShapeDtypeStruct((ADJ_ROWS, D), jnp.float32),
    )(trainable_table, A_w, A_b.reshape(1, INNER), B_w, B_b.reshape(1, D))


def _make_depad(vocab):
    n_tiles = vocab // 8
    per_w = (n_tiles // (NW * 2 * KT)) * (2 * KT)
    tail = n_tiles - per_w * NW
    n_blk = per_w // KT
    assert n_blk * KT == per_w and n_blk % 2 == 0
    assert tail % 8 == 0

    mesh = plsc.VectorSubcoreMesh(core_axis_name="c", subcore_axis_name="s")

    @functools.partial(
        pl.kernel,
        out_type=jax.ShapeDtypeStruct((vocab // 2, 128), jnp.float32),
        mesh=mesh,
        scratch_types=[
            pltpu.VMEM((KT, 8, D), jnp.float32),
            pltpu.VMEM((KT, 8, D), jnp.float32),
            pltpu.VMEM((4 * KT, 128), jnp.float32),
            pltpu.VMEM((4 * KT, 128), jnp.float32),
            pltpu.VMEM((8, 8, D), jnp.float32),
            pltpu.VMEM((ADJ_ROWS, D), jnp.float32),
            pltpu.SemaphoreType.DMA,
            pltpu.SemaphoreType.DMA,
            pltpu.SemaphoreType.DMA,
            pltpu.SemaphoreType.DMA,
        ],
    )
    def depad(table2d_hbm, adj_hbm, out_hbm, va, vb, vpa, vpb, vt, adj_v,
              sem_a, sem_b, sem_wa, sem_wb):
        wid = lax.axis_index("s") * NC + lax.axis_index("c")
        table_hbm = table2d_hbm.reshape(n_tiles, 8, D)
        t0 = wid * per_w

        def read(blk, dst, sem):
            pltpu.async_copy(table_hbm.at[pl.ds(t0 + blk * KT, KT)], dst, sem)

        def drain_r(dst, sem):
            pltpu.make_async_copy(table_hbm.at[pl.ds(0, KT)], dst, sem).wait()

        def drain_w(vp, sem):
            pltpu.make_async_copy(out_hbm.at[pl.ds(0, 4 * KT)], vp,
                                  sem).wait()

        def repack(src, vp):
            for t in range(KT):
                for s in range(8):
                    for c in range(0, D, LANES):
                        v = src[t, s, pl.ds(c, LANES)]
                        flat = (t * 8 + s) * D + c
                        vp[flat // 128, pl.ds(flat % 128, LANES)] = v

        def write(blk, vp, sem):
            pltpu.async_copy(
                vp, out_hbm.at[pl.ds((t0 + blk * KT) * 4, 4 * KT)], sem)

        read(0, va, sem_a)

        def pair_body(i, carry):
            blk = i * 2
            read(blk + 1, vb, sem_b)
            drain_r(va, sem_a)

            @pl.when(i > 0)
            def _():
                drain_w(vpa, sem_wa)
            repack(va, vpa)
            write(blk, vpa, sem_wa)

            @pl.when(blk + 2 < n_blk)
            def _():
                read(blk + 2, va, sem_a)
            drain_r(vb, sem_b)

            @pl.when(i > 0)
            def _():
                drain_w(vpb, sem_wb)
            repack(vb, vpb)
            write(blk + 1, vpb, sem_wb)
            return carry

        lax.fori_loop(0, n_blk // 2, pair_body, 0)
        drain_w(vpa, sem_wa)
        drain_w(vpb, sem_wb)

        if tail:
            @pl.when(wid == 0)
            def _tail():
                base = n_tiles - tail

                def tail_body(tb, carry):
                    pltpu.sync_copy(table_hbm.at[pl.ds(base + tb * 8, 8)], vt)
                    for t in range(8):
                        for s in range(8):
                            for c in range(0, D, LANES):
                                v = vt[t, s, pl.ds(c, LANES)]
                                flat = (t * 8 + s) * D + c
                                vpa[flat // 128, pl.ds(flat % 128, LANES)] = v
                    pltpu.sync_copy(vpa.at[pl.ds(0, 32)],
                                    out_hbm.at[pl.ds((base + tb * 8) * 4, 32)])
                    return carry

                lax.fori_loop(0, tail // 8, tail_body, 0)

        @pl.when(wid == 0)
        def _patch():
            pltpu.sync_copy(adj_hbm, adj_v)
            pltpu.sync_copy(out_hbm.at[pl.ds(0, N_TRAIN // 2)],
                            vpb.at[pl.ds(0, N_TRAIN // 2)])
            for q in range(N_TRAIN // 2):
                for h in range(2):
                    for c in range(0, D, LANES):
                        cur = vpb[q, pl.ds(h * D + c, LANES)]
                        addv = adj_v[2 * q + h, pl.ds(c, LANES)]
                        vpb[q, pl.ds(h * D + c, LANES)] = cur + addv
            pltpu.sync_copy(vpb.at[pl.ds(0, N_TRAIN // 2)],
                            out_hbm.at[pl.ds(0, N_TRAIN // 2)])

    return depad


def _make_sc_lookup(n_total, vocab, out_shape3):
    assert n_total % (NW * SUB) == 0
    per_w = n_total // NW
    chunk = 320
    assert per_w % chunk == 0 and chunk % SUB == 0
    n_chunks = per_w // chunk
    nsub = chunk // SUB
    nbuf = 4

    mesh = plsc.VectorSubcoreMesh(core_axis_name="c", subcore_axis_name="s")

    @functools.partial(
        pl.kernel,
        out_type=jax.ShapeDtypeStruct(out_shape3, jnp.float32),
        mesh=mesh,
        scratch_types=[
            pltpu.VMEM((chunk,), jnp.int32),
            pltpu.VMEM((chunk,), jnp.int32),
            pltpu.VMEM((SUB, 128), jnp.float32),
            pltpu.VMEM((SUB, 128), jnp.float32),
            pltpu.VMEM((SUB, 128), jnp.float32),
            pltpu.VMEM((SUB, 128), jnp.float32),
            pltpu.VMEM((chunk // 8, 8, D), jnp.float32),
            pltpu.SemaphoreType.DMA,
            pltpu.SemaphoreType.DMA,
            pltpu.SemaphoreType.DMA,
            pltpu.SemaphoreType.DMA,
        ],
    )
    def sc_lookup(ids_hbm, pairs_hbm, out3_hbm,
                  idx_v, pidx_v, pva, pvb, pvc, pvd, rows_v,
                  sem_a, sem_b, sem_c, sem_d):
        wid = lax.axis_index("s") * NC + lax.axis_index("c")
        out_hbm = out3_hbm.reshape(n_total // 8, 8, D)
        base_w = wid * per_w
        bufs = [(pva, sem_a), (pvb, sem_b), (pvc, sem_c), (pvd, sem_d)]

        def chunk_body(ch, carry):
            base = base_w + ch * chunk
            pltpu.sync_copy(ids_hbm.at[pl.ds(base, chunk)], idx_v)
            for q in range(chunk // LANES):
                iv = idx_v[pl.ds(q * LANES, LANES)]
                pidx_v[pl.ds(q * LANES, LANES)] = jnp.right_shift(
                    iv, jnp.int32(1))

            def fire(j):
                buf, sem = bufs[j % nbuf]
                return pltpu.async_copy(
                    pairs_hbm.at[pidx_v.at[pl.ds(j * SUB, SUB)]], buf, sem)

            cps = [fire(j) for j in range(min(nbuf, nsub))]

            for j in range(nsub):
                cps[j].wait()
                buf = bufs[j % nbuf][0]

                def egroup(k, _, j=j, buf=buf):
                    p0 = j * SUB + k * LANES
                    halfv = jnp.bitwise_and(idx_v[pl.ds(p0, LANES)],
                                            jnp.int32(1)) * D
                    for r in range(LANES):
                        half = halfv[r]
                        slot = k * LANES + r
                        prow = lax.div(p0 + r, 8)
                        for c in range(0, D, LANES):
                            rows_v[prow, r % 8, pl.ds(c, LANES)] = (
                                buf[slot, pl.ds(half + c, LANES)])
                    return 0
                lax.fori_loop(0, SUB // LANES, egroup, 0)
                if j + nbuf < nsub:
                    cps.append(fire(j + nbuf))

            pltpu.sync_copy(rows_v,
                            out_hbm.at[pl.ds(base // 8, chunk // 8)])
            return 0

        lax.fori_loop(0, n_chunks, chunk_body, 0)

    return sc_lookup


def kernel(input_ids, ids, orig_table, trainable_table, A_w, A_b, B_w, B_b):
    b, l = input_ids.shape
    n_total = b * l
    vocab = orig_table.shape[0]
    adj = _compute_adj_table(trainable_table, A_w, A_b, B_w, B_b)
    ids_flat = input_ids.astype(jnp.int32).reshape(n_total)
    pairs = _make_depad(vocab)(orig_table, adj)
    out = _make_sc_lookup(n_total, vocab, (b, l, D))(ids_flat, pairs)
    return out


# speedup vs baseline: 1.7159x; 1.1923x over previous
import functools

import jax
import jax.numpy as jnp
from jax import lax
from jax.experimental import pallas as pl
from jax.experimental.pallas import tpu as pltpu
from jax.experimental.pallas import tpu_sc as plsc

D = 64
INNER = 16
N_TRAIN = 64
ADJ_ROWS = 72

NC, NS, LANES = 2, 16, 16
NW = NC * NS
SUB = 64
KT = 32


def _adj_table_kernel(t_ref, aw_ref, ab_ref, bw_ref, bb_ref, out_ref):
    t = t_ref[...]
    a = lax.dot_general(t, aw_ref[...], (((1,), (1,)), ((), ())),
                        preferred_element_type=jnp.float32)
    a = a + ab_ref[...]
    adj = lax.dot_general(a, bw_ref[...], (((1,), (1,)), ((), ())),
                          preferred_element_type=jnp.float32)
    adj = adj + bb_ref[...]
    out_ref[...] = jnp.zeros_like(out_ref)
    out_ref[0:N_TRAIN, :] = adj


def _compute_adj_table(trainable_table, A_w, A_b, B_w, B_b):
    return pl.pallas_call(
        _adj_table_kernel,
        out_shape=jax.ShapeDtypeStruct((ADJ_ROWS, D), jnp.float32),
    )(trainable_table, A_w, A_b.reshape(1, INNER), B_w, B_b.reshape(1, D))


def _make_depad(vocab):
    n_tiles = vocab // 8
    per_w = (n_tiles // (NW * 2 * KT)) * (2 * KT)
    tail = n_tiles - per_w * NW
    n_blk = per_w // KT
    assert n_blk * KT == per_w and n_blk % 2 == 0
    assert tail % 8 == 0

    mesh = plsc.VectorSubcoreMesh(core_axis_name="c", subcore_axis_name="s")

    @functools.partial(
        pl.kernel,
        out_type=jax.ShapeDtypeStruct((vocab // 2, 128), jnp.float32),
        mesh=mesh,
        scratch_types=[
            pltpu.VMEM((KT, 8, D), jnp.float32),
            pltpu.VMEM((KT, 8, D), jnp.float32),
            pltpu.VMEM((4 * KT, 128), jnp.float32),
            pltpu.VMEM((4 * KT, 128), jnp.float32),
            pltpu.VMEM((8, 8, D), jnp.float32),
            pltpu.VMEM((ADJ_ROWS, D), jnp.float32),
            pltpu.SemaphoreType.DMA,
            pltpu.SemaphoreType.DMA,
            pltpu.SemaphoreType.DMA,
            pltpu.SemaphoreType.DMA,
        ],
    )
    def depad(table16_hbm, adj_hbm, out_hbm, va, vb, vpa, vpb, vt, adj_v,
              sem_a, sem_b, sem_wa, sem_wb):
        wid = lax.axis_index("s") * NC + lax.axis_index("c")
        table_hbm = table16_hbm.reshape(n_tiles, 8, D)
        t0 = wid * per_w

        def read(blk, dst, sem):
            pltpu.async_copy(table_hbm.at[pl.ds(t0 + blk * KT, KT)], dst, sem)

        def drain_r(dst, sem):
            pltpu.make_async_copy(table_hbm.at[pl.ds(0, KT)], dst, sem).wait()

        def drain_w(vp, sem):
            pltpu.make_async_copy(out_hbm.at[pl.ds(0, 4 * KT)], vp,
                                  sem).wait()

        def repack(src, vp):
            for t in range(KT):
                for s in range(8):
                    for c in range(0, D, LANES):
                        v = src[t, s, pl.ds(c, LANES)]
                        flat = (t * 8 + s) * D + c
                        vp[flat // 128, pl.ds(flat % 128, LANES)] = v

        def write(blk, vp, sem):
            pltpu.async_copy(
                vp, out_hbm.at[pl.ds((t0 + blk * KT) * 4, 4 * KT)], sem)

        read(0, va, sem_a)

        def pair_body(i, carry):
            blk = i * 2
            read(blk + 1, vb, sem_b)
            drain_r(va, sem_a)

            @pl.when(i > 0)
            def _():
                drain_w(vpa, sem_wa)
            repack(va, vpa)
            write(blk, vpa, sem_wa)

            @pl.when(blk + 2 < n_blk)
            def _():
                read(blk + 2, va, sem_a)
            drain_r(vb, sem_b)

            @pl.when(i > 0)
            def _():
                drain_w(vpb, sem_wb)
            repack(vb, vpb)
            write(blk + 1, vpb, sem_wb)
            return carry

        lax.fori_loop(0, n_blk // 2, pair_body, 0)
        drain_w(vpa, sem_wa)
        drain_w(vpb, sem_wb)

        if tail:
            @pl.when(wid == 0)
            def _tail():
                base = n_tiles - tail

                def tail_body(tb, carry):
                    pltpu.sync_copy(table_hbm.at[pl.ds(base + tb * 8, 8)], vt)
                    for t in range(8):
                        for s in range(8):
                            for c in range(0, D, LANES):
                                v = vt[t, s, pl.ds(c, LANES)]
                                flat = (t * 8 + s) * D + c
                                vpa[flat // 128, pl.ds(flat % 128, LANES)] = v
                    pltpu.sync_copy(vpa.at[pl.ds(0, 32)],
                                    out_hbm.at[pl.ds((base + tb * 8) * 4, 32)])
                    return carry

                lax.fori_loop(0, tail // 8, tail_body, 0)

        @pl.when(wid == 0)
        def _patch():
            pltpu.sync_copy(adj_hbm, adj_v)
            pltpu.sync_copy(out_hbm.at[pl.ds(0, N_TRAIN // 2)],
                            vpb.at[pl.ds(0, N_TRAIN // 2)])
            for q in range(N_TRAIN // 2):
                for h in range(2):
                    for c in range(0, D, LANES):
                        cur = vpb[q, pl.ds(h * D + c, LANES)]
                        addv = adj_v[2 * q + h, pl.ds(c, LANES)]
                        vpb[q, pl.ds(h * D + c, LANES)] = cur + addv
            pltpu.sync_copy(vpb.at[pl.ds(0, N_TRAIN // 2)],
                            out_hbm.at[pl.ds(0, N_TRAIN // 2)])

    return depad


def _make_sc_lookup(n_total, vocab):
    assert n_total % (NW * SUB) == 0
    per_w = n_total // NW
    chunk = 320
    assert per_w % chunk == 0 and chunk % SUB == 0
    n_chunks = per_w // chunk
    nsub = chunk // SUB
    nbuf = 4

    mesh = plsc.VectorSubcoreMesh(core_axis_name="c", subcore_axis_name="s")

    @functools.partial(
        pl.kernel,
        out_type=jax.ShapeDtypeStruct((n_total // 16, 16, D), jnp.float32),
        mesh=mesh,
        scratch_types=[
            pltpu.VMEM((chunk,), jnp.int32),
            pltpu.VMEM((chunk,), jnp.int32),
            pltpu.VMEM((SUB, 128), jnp.float32),
            pltpu.VMEM((SUB, 128), jnp.float32),
            pltpu.VMEM((SUB, 128), jnp.float32),
            pltpu.VMEM((SUB, 128), jnp.float32),
            pltpu.VMEM((chunk // 8, 8, D), jnp.float32),
            pltpu.SemaphoreType.DMA,
            pltpu.SemaphoreType.DMA,
            pltpu.SemaphoreType.DMA,
            pltpu.SemaphoreType.DMA,
        ],
    )
    def sc_lookup(ids_hbm, pairs_hbm, out3_hbm,
                  idx_v, pidx_v, pva, pvb, pvc, pvd, rows_v,
                  sem_a, sem_b, sem_c, sem_d):
        wid = lax.axis_index("s") * NC + lax.axis_index("c")
        out_hbm = out3_hbm.reshape(n_total // 8, 8, D)
        base_w = wid * per_w
        bufs = [(pva, sem_a), (pvb, sem_b), (pvc, sem_c), (pvd, sem_d)]

        def chunk_body(ch, carry):
            base = base_w + ch * chunk
            pltpu.sync_copy(ids_hbm.at[pl.ds(base, chunk)], idx_v)
            for q in range(chunk // LANES):
                iv = idx_v[pl.ds(q * LANES, LANES)]
                pidx_v[pl.ds(q * LANES, LANES)] = jnp.right_shift(
                    iv, jnp.int32(1))

            def fire(j):
                buf, sem = bufs[j % nbuf]
                return pltpu.async_copy(
                    pairs_hbm.at[pidx_v.at[pl.ds(j * SUB, SUB)]], buf, sem)

            cps = [fire(j) for j in range(min(nbuf, nsub))]

            for j in range(nsub):
                cps[j].wait()
                buf = bufs[j % nbuf][0]

                def egroup(k, _, j=j, buf=buf):
                    p0 = j * SUB + k * LANES
                    halfv = jnp.bitwise_and(idx_v[pl.ds(p0, LANES)],
                                            jnp.int32(1)) * D
                    for r in range(LANES):
                        half = halfv[r]
                        slot = k * LANES + r
                        prow = lax.div(p0 + r, 8)
                        for c in range(0, D, LANES):
                            rows_v[prow, r % 8, pl.ds(c, LANES)] = (
                                buf[slot, pl.ds(half + c, LANES)])
                    return 0
                lax.fori_loop(0, SUB // LANES, egroup, 0)
                if j + nbuf < nsub:
                    cps.append(fire(j + nbuf))

            pltpu.sync_copy(rows_v,
                            out_hbm.at[pl.ds(base // 8, chunk // 8)])
            return 0

        lax.fori_loop(0, n_chunks, chunk_body, 0)

    return sc_lookup


def kernel(input_ids, ids, orig_table, trainable_table, A_w, A_b, B_w, B_b):
    b, l = input_ids.shape
    n_total = b * l
    vocab = orig_table.shape[0]
    adj = _compute_adj_table(trainable_table, A_w, A_b, B_w, B_b)
    ids_flat = input_ids.astype(jnp.int32).reshape(n_total)
    table16 = orig_table.reshape(vocab // 16, 16, D)
    pairs = _make_depad(vocab)(table16, adj)
    out = _make_sc_lookup(n_total, vocab)(ids_flat, pairs)
    return out.reshape(b, l, D)
